# baseline (speedup 1.0000x reference)
import functools

import jax
import jax.numpy as jnp
from jax import lax
from jax.experimental import pallas as pl
from jax.experimental.pallas import tpu as pltpu
from jax.experimental.pallas import tpu_sc as plsc

_N_HEADS = 32
_MAX_CTX = 8192
_HDIM = 128
_QLEN = 16


def _sc_body(kc_hbm, vc_hbm, pos_hbm, k_hbm, v_hbm, ko_hbm, vo_hbm,
             idx_v, krows_v, vrows_v, sem_k, sem_v):
    wid = lax.axis_index("s") * 2 + lax.axis_index("c")
    row0 = wid * _MAX_CTX
    pltpu.sync_copy(pos_hbm, idx_v)
    idx_v[...] = idx_v[...] + row0
    pltpu.sync_copy(k_hbm.at[pl.ds(wid * _QLEN, _QLEN)], krows_v)
    pltpu.sync_copy(v_hbm.at[pl.ds(wid * _QLEN, _QLEN)], vrows_v)
    pltpu.sync_copy(kc_hbm.at[pl.ds(row0, _MAX_CTX)],
                    ko_hbm.at[pl.ds(row0, _MAX_CTX)])
    pltpu.sync_copy(vc_hbm.at[pl.ds(row0, _MAX_CTX)],
                    vo_hbm.at[pl.ds(row0, _MAX_CTX)])
    ck = pltpu.make_async_copy(krows_v, ko_hbm.at[idx_v], sem_k)
    cv = pltpu.make_async_copy(vrows_v, vo_hbm.at[idx_v], sem_v)
    ck.start()
    cv.start()
    ck.wait()
    cv.wait()


def kernel(k_cache, v_cache, pos_ids, k, v):
    kc2 = k_cache.reshape(_N_HEADS * _MAX_CTX, _HDIM)
    vc2 = v_cache.reshape(_N_HEADS * _MAX_CTX, _HDIM)
    k2 = k.reshape(_N_HEADS * _QLEN, _HDIM)
    v2 = v.reshape(_N_HEADS * _QLEN, _HDIM)
    pos = pos_ids.astype(jnp.int32)

    mesh = plsc.VectorSubcoreMesh(core_axis_name="c", subcore_axis_name="s")
    run = functools.partial(
        pl.kernel,
        out_type=[
            jax.ShapeDtypeStruct((_N_HEADS * _MAX_CTX, _HDIM), jnp.float32),
            jax.ShapeDtypeStruct((_N_HEADS * _MAX_CTX, _HDIM), jnp.float32),
        ],
        mesh=mesh,
        scratch_types=[
            pltpu.VMEM((_QLEN,), jnp.int32),
            pltpu.VMEM((_QLEN, _HDIM), jnp.float32),
            pltpu.VMEM((_QLEN, _HDIM), jnp.float32),
            pltpu.SemaphoreType.DMA,
            pltpu.SemaphoreType.DMA,
        ],
    )(_sc_body)
    ko, vo = run(kc2, vc2, pos, k2, v2)
    return (ko.reshape(k_cache.shape), vo.reshape(v_cache.shape))


# speedup vs baseline: 45.1617x; 45.1617x over previous
import jax
import jax.numpy as jnp
from jax.experimental import pallas as pl
from jax.experimental.pallas import tpu as pltpu

_N_HEADS = 32
_MAX_CTX = 8192
_HDIM = 128
_QLEN = 16
_WIN = 8


def _scatter_kernel(pos_ref, kc_ref, vc_ref, k_ref, v_ref, ko_ref, vo_ref):
    i = pl.program_id(0)
    w = pos_ref[i] // _WIN
    ko_ref[...] = kc_ref[...]
    vo_ref[...] = vc_ref[...]
    for j in range(_QLEN):
        pj = pos_ref[j]

        @pl.when(pj // _WIN == w)
        def _():
            r = pj % _WIN
            ko_ref[0, :, r, :] = k_ref[0, :, j, :]
            vo_ref[0, :, r, :] = v_ref[0, :, j, :]


def kernel(k_cache, v_cache, pos_ids, k, v):
    win_spec = pl.BlockSpec((1, _N_HEADS, _WIN, _HDIM),
                            lambda i, pos_ref: (0, 0, pos_ref[i] // _WIN, 0))
    kv_spec = pl.BlockSpec((1, _N_HEADS, _QLEN, _HDIM),
                           lambda i, pos_ref: (0, 0, 0, 0))
    ko, vo = pl.pallas_call(
        _scatter_kernel,
        grid_spec=pltpu.PrefetchScalarGridSpec(
            num_scalar_prefetch=1,
            grid=(_QLEN,),
            in_specs=[win_spec, win_spec, kv_spec, kv_spec],
            out_specs=[win_spec, win_spec],
        ),
        out_shape=[
            jax.ShapeDtypeStruct(k_cache.shape, k_cache.dtype),
            jax.ShapeDtypeStruct(v_cache.shape, v_cache.dtype),
        ],
        input_output_aliases={1: 0, 2: 1},
    )(pos_ids.astype(jnp.int32), k_cache, v_cache, k, v)
    return (ko, vo)


# speedup vs baseline: 48.7007x; 1.0784x over previous
import jax
import jax.numpy as jnp
from jax.experimental import pallas as pl
from jax.experimental.pallas import tpu as pltpu

_N_HEADS = 32
_MAX_CTX = 8192
_HDIM = 128
_QLEN = 16
_BLK = 8192
_NBLK = _MAX_CTX // _BLK


def _merge_kernel(pos_ref, kc_ref, vc_ref, k_ref, v_ref, ko_ref, vo_ref):
    ko_ref[...] = kc_ref[...]
    vo_ref[...] = vc_ref[...]
    base = pl.program_id(1) * _BLK
    for i in range(_QLEN):
        p = pos_ref[i]

        @pl.when(jnp.logical_and(p >= base, p < base + _BLK))
        def _():
            ko_ref[0, 0, p - base, :] = k_ref[0, 0, i, :]
            vo_ref[0, 0, p - base, :] = v_ref[0, 0, i, :]


def kernel(k_cache, v_cache, pos_ids, k, v):
    cache_spec = pl.BlockSpec((1, 1, _BLK, _HDIM), lambda h, j: (0, h, j, 0))
    kv_spec = pl.BlockSpec((1, 1, _QLEN, _HDIM), lambda h, j: (0, h, 0, 0))
    pos_spec = pl.BlockSpec(memory_space=pltpu.SMEM)
    ko, vo = pl.pallas_call(
        _merge_kernel,
        grid=(_N_HEADS, _NBLK),
        in_specs=[pos_spec, cache_spec, cache_spec, kv_spec, kv_spec],
        out_specs=[cache_spec, cache_spec],
        out_shape=[
            jax.ShapeDtypeStruct(k_cache.shape, k_cache.dtype),
            jax.ShapeDtypeStruct(v_cache.shape, v_cache.dtype),
        ],
        compiler_params=pltpu.CompilerParams(
            dimension_semantics=("parallel", "parallel"),
        ),
    )(pos_ids.astype(jnp.int32), k_cache, v_cache, k, v)
    return (ko, vo)
